# baseline (speedup 1.0000x reference)
import functools
import math

import jax
import jax.numpy as jnp
from jax import lax
from jax.experimental import pallas as pl
from jax.experimental.pallas import tpu as pltpu
from jax.experimental.pallas import tpu_sc as plsc

SMOOTH = 0.1
CONF = 1.0 - SMOOTH

RED_ROWS = 25000
RED_COLS = 4096
BLK_ROWS = 200
GRID = RED_ROWS // BLK_ROWS

NC = 2
NS = 16
NW = NC * NS
LANES = 16


def _sc_gather_partials(x1, target, size):
    n = target.shape[0]
    b_per_w = n // NW
    n_chunks = b_per_w // LANES
    mesh = plsc.VectorSubcoreMesh(core_axis_name="c", subcore_axis_name="s")

    @functools.partial(
        pl.kernel,
        mesh=mesh,
        out_type=jax.ShapeDtypeStruct((NW, LANES), jnp.float32),
        scratch_types=[
            pltpu.VMEM((b_per_w,), jnp.int32),
            pltpu.VMEM((b_per_w,), jnp.int32),
            pltpu.VMEM((b_per_w,), jnp.float32),
            pltpu.VMEM((LANES,), jnp.float32),
            pltpu.SemaphoreType.DMA,
        ],
    )
    def sc_kernel(x_hbm, tgt_hbm, out_hbm, tgt_v, idx_v, val_v, acc_v, sem):
        wid = lax.axis_index("s") * NC + lax.axis_index("c")
        base = wid * b_per_w
        pltpu.sync_copy(tgt_hbm.at[pl.ds(base, b_per_w)], tgt_v)
        for j in range(n_chunks):
            t = tgt_v[pl.ds(j * LANES, LANES)]
            ivec = lax.iota(jnp.int32, LANES) + (base + j * LANES)
            idx_v[pl.ds(j * LANES, LANES)] = ivec * size + t
        pltpu.async_copy(x_hbm.at[idx_v], val_v, sem).wait()
        acc = jnp.zeros((LANES,), jnp.float32)
        for j in range(n_chunks):
            acc = acc + val_v[pl.ds(j * LANES, LANES)]
        acc_v[...] = acc
        pltpu.sync_copy(acc_v, out_hbm.at[wid])

    return sc_kernel(x1, target)


def kernel(x, target):
    n, size = x.shape
    s = SMOOTH / (size - 1)
    c0 = n * ((size - 1) * s * math.log(s) + CONF * math.log(CONF))
    s_coef = float(s)
    g_coef = float(CONF - s)
    c0 = float(c0)

    g_part = _sc_gather_partials(x.reshape(-1), target.astype(jnp.int32),
                                 size)

    xr = x.reshape(RED_ROWS, RED_COLS)

    def tc_body(x_ref, g_ref, o_ref):
        i = pl.program_id(0)

        @pl.when(i == 0)
        def _init():
            o_ref[0, 0] = jnp.float32(0.0)

        o_ref[0, 0] += jnp.sum(x_ref[...])

        @pl.when(i == GRID - 1)
        def _final():
            o_ref[0, 0] = (c0 - s_coef * o_ref[0, 0]
                           - g_coef * jnp.sum(g_ref[...]))

    out = pl.pallas_call(
        tc_body,
        grid=(GRID,),
        in_specs=[
            pl.BlockSpec((BLK_ROWS, RED_COLS), lambda i: (i, 0)),
            pl.BlockSpec((NW, LANES), lambda i: (0, 0)),
        ],
        out_specs=pl.BlockSpec((1, 1), lambda i: (0, 0),
                               memory_space=pltpu.SMEM),
        out_shape=jax.ShapeDtypeStruct((1, 1), jnp.float32),
    )(xr, g_part)
    return out.reshape(())


# speedup vs baseline: 2.7998x; 2.7998x over previous
import functools
import math

import jax
import jax.numpy as jnp
from jax import lax
from jax.experimental import pallas as pl
from jax.experimental.pallas import tpu as pltpu
from jax.experimental.pallas import tpu_sc as plsc

SMOOTH = 0.1
CONF = 1.0 - SMOOTH

BLK_ROWS = 16

NC = 2
NS = 16
NW = NC * NS
LANES = 16


def _sc_gather_partials(x, target):
    n = target.shape[0]
    b_per_w = n // NW
    n_chunks = b_per_w // LANES
    mesh = plsc.VectorSubcoreMesh(core_axis_name="c", subcore_axis_name="s")

    @functools.partial(
        pl.kernel,
        mesh=mesh,
        out_type=jax.ShapeDtypeStruct((NW, LANES), jnp.float32),
        scratch_types=[
            pltpu.VMEM((b_per_w,), jnp.int32),
            pltpu.VMEM((b_per_w, 8, 128), jnp.float32),
            pltpu.VMEM((LANES,), jnp.float32),
            pltpu.SemaphoreType.DMA,
        ],
    )
    def sc_kernel(x_hbm, tgt_hbm, out_hbm, tgt_v, tile_v, acc_v, sem):
        wid = lax.axis_index("s") * NC + lax.axis_index("c")
        base = wid * b_per_w
        pltpu.sync_copy(tgt_hbm.at[pl.ds(base, b_per_w)], tgt_v)
        lane_iota = lax.iota(jnp.int32, LANES)
        t_es = []
        copies = []
        tchunk = [tgt_v[pl.ds(j * LANES, LANES)] for j in range(n_chunks)]
        for e in range(b_per_w):
            j, l = divmod(e, LANES)
            t_e = tchunk[j][l]
            t_es.append(t_e)
            row0 = pl.multiple_of(base + (e & ~7), 8)
            col0 = pl.multiple_of(t_e & ~127, 128)
            copies.append(pltpu.async_copy(
                x_hbm.at[pl.ds(row0, 8), pl.ds(col0, 128)], tile_v.at[e],
                sem))
        for cp in copies:
            cp.wait()
        acc = jnp.zeros((LANES,), jnp.float32)
        for e in range(b_per_w):
            t_e = t_es[e]
            chunk0 = t_e & 112
            lane_e = t_e & (LANES - 1)
            seg = tile_v[e, e & 7, pl.ds(chunk0, LANES)]
            acc = acc + jnp.where(lane_iota == lane_e, seg, 0.0)
        acc_v[...] = acc
        pltpu.sync_copy(acc_v, out_hbm.at[wid])

    return sc_kernel(x, target)


def kernel(x, target):
    n, size = x.shape
    s = SMOOTH / (size - 1)
    c0 = float(n * ((size - 1) * s * math.log(s) + CONF * math.log(CONF)))
    s_coef = float(s)
    g_coef = float(CONF - s)
    grid = n // BLK_ROWS

    g_part = _sc_gather_partials(x, target.astype(jnp.int32))

    def tc_body(x_ref, g_ref, o_ref):
        i = pl.program_id(0)

        @pl.when(i == 0)
        def _init():
            o_ref[0, 0] = jnp.float32(0.0)

        o_ref[0, 0] += jnp.sum(x_ref[...])

        @pl.when(i == grid - 1)
        def _final():
            o_ref[0, 0] = (c0 - s_coef * o_ref[0, 0]
                           - g_coef * jnp.sum(g_ref[...]))

    out = pl.pallas_call(
        tc_body,
        grid=(grid,),
        in_specs=[
            pl.BlockSpec((BLK_ROWS, size), lambda i: (i, 0)),
            pl.BlockSpec((NW, LANES), lambda i: (0, 0)),
        ],
        out_specs=pl.BlockSpec((1, 1), lambda i: (0, 0),
                               memory_space=pltpu.SMEM),
        out_shape=jax.ShapeDtypeStruct((1, 1), jnp.float32),
    )(x, g_part)
    return out.reshape(())


# speedup vs baseline: 2.9796x; 1.0642x over previous
import functools
import math

import jax
import jax.numpy as jnp
from jax import lax
from jax.experimental import pallas as pl
from jax.experimental.pallas import tpu as pltpu
from jax.experimental.pallas import tpu_sc as plsc

SMOOTH = 0.1
CONF = 1.0 - SMOOTH

BLK_ROWS = 8

NC = 2
NS = 16
NW = NC * NS
LANES = 16


def _sc_gather_partials(x, target):
    n = target.shape[0]
    b_per_w = n // NW
    n_chunks = b_per_w // LANES
    mesh = plsc.VectorSubcoreMesh(core_axis_name="c", subcore_axis_name="s")

    @functools.partial(
        pl.kernel,
        mesh=mesh,
        out_type=jax.ShapeDtypeStruct((NW, LANES), jnp.float32),
        scratch_types=[
            pltpu.VMEM((b_per_w,), jnp.int32),
            pltpu.VMEM((b_per_w, 8, 128), jnp.float32),
            pltpu.VMEM((LANES,), jnp.float32),
            pltpu.SemaphoreType.DMA,
        ],
    )
    def sc_kernel(x_hbm, tgt_hbm, out_hbm, tgt_v, tile_v, acc_v, sem):
        wid = lax.axis_index("s") * NC + lax.axis_index("c")
        base = wid * b_per_w
        pltpu.sync_copy(tgt_hbm.at[pl.ds(base, b_per_w)], tgt_v)
        lane_iota = lax.iota(jnp.int32, LANES)
        t_es = []
        copies = []
        tchunk = [tgt_v[pl.ds(j * LANES, LANES)] for j in range(n_chunks)]
        for e in range(b_per_w):
            j, l = divmod(e, LANES)
            t_e = tchunk[j][l]
            t_es.append(t_e)
            row0 = pl.multiple_of(base + (e & ~7), 8)
            col0 = pl.multiple_of(t_e & ~127, 128)
            copies.append(pltpu.async_copy(
                x_hbm.at[pl.ds(row0, 8), pl.ds(col0, 128)], tile_v.at[e],
                sem))
        for cp in copies:
            cp.wait()
        acc = jnp.zeros((LANES,), jnp.float32)
        for e in range(b_per_w):
            t_e = t_es[e]
            chunk0 = t_e & 112
            lane_e = t_e & (LANES - 1)
            seg = tile_v[e, e & 7, pl.ds(chunk0, LANES)]
            acc = acc + jnp.where(lane_iota == lane_e, seg, 0.0)
        acc_v[...] = acc
        pltpu.sync_copy(acc_v, out_hbm.at[wid])

    return sc_kernel(x, target)


def kernel(x, target):
    n, size = x.shape
    s = SMOOTH / (size - 1)
    c0 = float(n * ((size - 1) * s * math.log(s) + CONF * math.log(CONF)))
    s_coef = float(s)
    g_coef = float(CONF - s)

    g_part = _sc_gather_partials(x, target.astype(jnp.int32))

    nstreams = 4
    grid = n // (BLK_ROWS * nstreams)

    def tc_body(*refs):
        x_refs = refs[:nstreams]
        g_ref, o_ref = refs[nstreams], refs[nstreams + 1]
        i = pl.program_id(0)

        @pl.when(i == 0)
        def _init():
            o_ref[0, 0] = jnp.float32(0.0)

        part = x_refs[0][...]
        for r in x_refs[1:]:
            part = part + r[...]
        o_ref[0, 0] += jnp.sum(part)

        @pl.when(i == grid - 1)
        def _final():
            o_ref[0, 0] = (c0 - s_coef * o_ref[0, 0]
                           - g_coef * jnp.sum(g_ref[...]))

    out = pl.pallas_call(
        tc_body,
        grid=(grid,),
        in_specs=[
            pl.BlockSpec((BLK_ROWS, size),
                         functools.partial(lambda k, i: (nstreams * i + k, 0),
                                           k))
            for k in range(nstreams)
        ] + [
            pl.BlockSpec((NW, LANES), lambda i: (0, 0)),
        ],
        out_specs=pl.BlockSpec((1, 1), lambda i: (0, 0),
                               memory_space=pltpu.SMEM),
        out_shape=jax.ShapeDtypeStruct((1, 1), jnp.float32),
    )(*([x] * nstreams), g_part)
    return out.reshape(())
